# baseline (speedup 1.0000x reference)
import functools

import jax
import jax.numpy as jnp
from jax import lax
from jax.experimental import pallas as pl
from jax.experimental.pallas import tpu as pltpu
import jax.experimental.pallas.tpu_sc as plsc

_N = 65536 * 6 * 7 * 8
_T_ELEMS = 2 * 6 * 7 * 8
_NC, _NS = 2, 16
_NW = _NC * _NS
_PER_W = _N // _NW
_CHUNK = 98304
_NCHUNKS = _PER_W // _CHUNK


def _t_kernel(t_ref):
    flat = (jax.lax.broadcasted_iota(jnp.int32, (2, _T_ELEMS // 2), 0)
            * (_T_ELEMS // 2)
            + jax.lax.broadcasted_iota(jnp.int32, (2, _T_ELEMS // 2), 1))
    t_ref[...] = flat.astype(jnp.float32)


@functools.partial(
    pl.kernel,
    out_type=jax.ShapeDtypeStruct((_N,), jnp.float32),
    mesh=plsc.VectorSubcoreMesh(core_axis_name="c", subcore_axis_name="s"),
    scratch_types=[
        pltpu.VMEM((_CHUNK,), jnp.float32),
        pltpu.VMEM((_T_ELEMS,), jnp.float32),
    ],
)
def _sc_copy(x_hbm, t_hbm, o_hbm, buf, tbuf):
    wid = lax.axis_index("s") * _NC + lax.axis_index("c")
    base = wid * _PER_W
    for i in range(_NCHUNKS):
        off = base + i * _CHUNK
        pltpu.sync_copy(x_hbm.at[pl.ds(off, _CHUNK)], buf)
        pltpu.sync_copy(buf, o_hbm.at[pl.ds(off, _CHUNK)])

    @pl.when(wid == 0)
    def _():
        pltpu.sync_copy(t_hbm, tbuf)
        pltpu.sync_copy(tbuf, o_hbm.at[pl.ds(0, _T_ELEMS)])


def kernel(x):
    t = pl.pallas_call(
        _t_kernel,
        out_shape=jax.ShapeDtypeStruct((2, _T_ELEMS // 2), jnp.float32),
    )().reshape(_T_ELEMS)
    out = _sc_copy(x.reshape(_N), t)
    return out.reshape(x.shape)


# speedup vs baseline: 11.1194x; 11.1194x over previous
import jax
import jax.numpy as jnp
from jax.experimental import pallas as pl
from jax.experimental.pallas import tpu as pltpu

_ROWS = 65536
_D = 6 * 7 * 8
_CHR = 2048
_NCH = _ROWS // _CHR
_K = 8
_LEAD = 4


def _ring_kernel(x_hbm, o_hbm, bufs, tbuf, sem_in, sem_out, sem_t):
    def mk_in(c):
        s = c % _K
        return pltpu.make_async_copy(
            x_hbm.at[pl.ds(c * _CHR, _CHR)], bufs.at[s], sem_in.at[s])

    def mk_out(c):
        s = c % _K
        return pltpu.make_async_copy(
            bufs.at[s], o_hbm.at[pl.ds(c * _CHR, _CHR)], sem_out.at[s])

    for c in range(_NCH + _K):
        if 0 <= c - _K < _NCH:
            mk_out(c - _K).wait()
        if c < _NCH:
            mk_in(c).start()
        d = c - _LEAD
        if 0 <= d < _NCH:
            mk_in(d).wait()
            mk_out(d).start()

    flat = (jax.lax.broadcasted_iota(jnp.int32, (2, _D), 0) * _D
            + jax.lax.broadcasted_iota(jnp.int32, (2, _D), 1))
    tbuf[...] = flat.astype(jnp.float32)
    small = pltpu.make_async_copy(tbuf, o_hbm.at[pl.ds(0, 2)], sem_t)
    small.start()
    small.wait()


def kernel(x):
    xf = x.reshape(_ROWS, _D)
    out = pl.pallas_call(
        _ring_kernel,
        in_specs=[pl.BlockSpec(memory_space=pltpu.MemorySpace.HBM)],
        out_specs=pl.BlockSpec(memory_space=pltpu.MemorySpace.HBM),
        out_shape=jax.ShapeDtypeStruct((_ROWS, _D), jnp.float32),
        scratch_shapes=[
            pltpu.VMEM((_K, _CHR, _D), jnp.float32),
            pltpu.VMEM((2, _D), jnp.float32),
            pltpu.SemaphoreType.DMA((_K,)),
            pltpu.SemaphoreType.DMA((_K,)),
            pltpu.SemaphoreType.DMA,
        ],
    )(xf)
    return out.reshape(x.shape)
